# Initial kernel scaffold; baseline (speedup 1.0000x reference)
import jax
import jax.numpy as jnp
from jax.experimental import pallas as pl


def kernel(x, edge_index, edge_attr, params):
    raise NotImplementedError("write your pallas kernel here")


# speedup vs baseline: 9.8489x; 9.8489x over previous
import functools

import jax
import jax.numpy as jnp
from jax import lax
from jax.experimental import pallas as pl
from jax.experimental.pallas import tpu as pltpu
from jax.experimental.pallas import tpu_sc as plsc

_N = 10000
_E = 320000
_D = 128
_EDIM = 16
_HEADS = [8, 8, 1]
_C = 128
_NEG = 0.2
_EPS = 1e-5
_HP = 16
_E1 = _E + _N
_NW = 32
_B = 128
_EP = 331776
_EW = _EP // _NW
_NBLK = _EW // _B
_NP = 10240
_STR = _NP // 16

_f32 = jnp.float32
_i32 = jnp.int32


def _tk_in(x, w, b):
    bn = 2000

    def body(x_ref, w_ref, b_ref, o_ref):
        o_ref[...] = jnp.dot(x_ref[...], w_ref[...],
                             preferred_element_type=_f32) + b_ref[...]

    return pl.pallas_call(
        body,
        grid=(_N // bn,),
        in_specs=[
            pl.BlockSpec((bn, _D), lambda i: (i, 0)),
            pl.BlockSpec((_D, _C), lambda i: (0, 0)),
            pl.BlockSpec((1, _C), lambda i: (0, 0)),
        ],
        out_specs=pl.BlockSpec((bn, _C), lambda i: (i, 0)),
        out_shape=jax.ShapeDtypeStruct((_N, _C), _f32),
    )(x, w, b.reshape(1, _C))


def _tk_proj(h, hcur, W, a_s, a_d):
    bn = 1000
    nb = _N // bn

    def body(h_ref, w_ref, as_ref, ad_ref, xw_ref, s_ref, d_ref):
        k = pl.program_id(1)
        xb = jnp.dot(h_ref[...], w_ref[...], preferred_element_type=_f32)
        xw_ref[...] = xb
        cs = jnp.sum(xb * as_ref[0], axis=1)
        cd = jnp.sum(xb * ad_ref[0], axis=1)
        lane = lax.broadcasted_iota(_i32, (bn, _HP), 1)
        ms = jnp.where(lane == k, cs[:, None], 0.0)
        md = jnp.where(lane == k, cd[:, None], 0.0)

        @pl.when(k == 0)
        def _():
            s_ref[...] = ms
            d_ref[...] = md

        @pl.when(k > 0)
        def _():
            s_ref[...] += ms
            d_ref[...] += md

    return pl.pallas_call(
        body,
        grid=(nb, h),
        in_specs=[
            pl.BlockSpec((bn, _C), lambda i, k: (i, 0)),
            pl.BlockSpec((_C, _C), lambda i, k: (0, k)),
            pl.BlockSpec((1, 1, _C), lambda i, k: (k, 0, 0)),
            pl.BlockSpec((1, 1, _C), lambda i, k: (k, 0, 0)),
        ],
        out_specs=[
            pl.BlockSpec((bn, _C), lambda i, k: (i, k)),
            pl.BlockSpec((bn, _HP), lambda i, k: (i, 0)),
            pl.BlockSpec((bn, _HP), lambda i, k: (i, 0)),
        ],
        out_shape=[
            jax.ShapeDtypeStruct((_N, h * _C), _f32),
            jax.ShapeDtypeStruct((_N, _HP), _f32),
            jax.ShapeDtypeStruct((_N, _HP), _f32),
        ],
    )(hcur, W, a_s.reshape(h, 1, _C), a_d.reshape(h, 1, _C))


def _tk_edge(h, edge_attr, We, a_e):
    be = 2000
    nbe = _E // be

    def body(ea_ref, we_ref, ae_ref, al_ref, es_ref, sf_ref):
        i = pl.program_id(0)
        Ae = jnp.sum(we_ref[...].reshape(_EDIM, h, _C) * ae_ref[...][None],
                     axis=-1)
        ab = jnp.dot(ea_ref[...], Ae, preferred_element_type=_f32)
        if h < _HP:
            ab = jnp.concatenate([ab, jnp.zeros((be, _HP - h), _f32)], axis=1)
        al_ref[...] = ab
        s = jnp.sum(ea_ref[...], axis=0)[None]

        @pl.when(i == 0)
        def _():
            es_ref[...] = s

        @pl.when(i > 0)
        def _():
            es_ref[...] += s

        @pl.when(i == nbe - 1)
        def _():
            m = es_ref[...] / float(_E)
            aes = jnp.dot(m, Ae, preferred_element_type=_f32)
            if h < _HP:
                aes = jnp.concatenate(
                    [aes, jnp.zeros((1, _HP - h), _f32)], axis=1)
            sf_ref[...] = aes

    return pl.pallas_call(
        body,
        grid=(nbe,),
        in_specs=[
            pl.BlockSpec((be, _EDIM), lambda i: (i, 0)),
            pl.BlockSpec((_EDIM, h * _C), lambda i: (0, 0)),
            pl.BlockSpec((h, _C), lambda i: (0, 0)),
        ],
        out_specs=[
            pl.BlockSpec((be, _HP), lambda i: (i, 0)),
            pl.BlockSpec((1, _EDIM), lambda i: (0, 0)),
            pl.BlockSpec((1, _HP), lambda i: (0, 0)),
        ],
        out_shape=[
            jax.ShapeDtypeStruct((_E, _HP), _f32),
            jax.ShapeDtypeStruct((1, _EDIM), _f32),
            jax.ShapeDtypeStruct((1, _HP), _f32),
        ],
    )(edge_attr, We, a_e)


def _tk_final1(h, num_p, den_p, b):
    bn = 1280
    nb = _NP // bn
    inv_h = 1.0 / h

    def body(n0_ref, n1_ref, d0_ref, d1_ref, b_ref, y_ref, s_ref):
        i = pl.program_id(0)
        k = pl.program_id(1)
        acc = n0_ref[...] + n1_ref[...]
        d = d0_ref[...] + d1_ref[...]
        lane = lax.broadcasted_iota(_i32, (bn, _HP), 1)
        dk = jnp.sum(jnp.where(lane == k, d, 0.0), axis=1)
        contrib = acc * (inv_h / (dk + 1e-16))[:, None]

        @pl.when(k == 0)
        def _():
            y_ref[...] = contrib

        @pl.when(k > 0)
        def _():
            y_ref[...] += contrib

        @pl.when(k == h - 1)
        def _():
            yf = y_ref[...] + b_ref[...]
            y_ref[...] = yf
            row = lax.broadcasted_iota(_i32, (bn, _C), 0) + i * bn
            ym = jnp.where(row < _N, yf, 0.0)
            s = jnp.concatenate([jnp.sum(ym, axis=0)[None],
                                 jnp.sum(ym * ym, axis=0)[None]], axis=0)

            @pl.when(i == 0)
            def _():
                s_ref[...] = s

            @pl.when(i > 0)
            def _():
                s_ref[...] += s

    return pl.pallas_call(
        body,
        grid=(nb, h),
        in_specs=[
            pl.BlockSpec((bn, _C), lambda i, k: (k * nb + i, 0)),
            pl.BlockSpec((bn, _C), lambda i, k: ((h + k) * nb + i, 0)),
            pl.BlockSpec((bn, _HP), lambda i, k: (i, 0)),
            pl.BlockSpec((bn, _HP), lambda i, k: (nb + i, 0)),
            pl.BlockSpec((1, _C), lambda i, k: (0, 0)),
        ],
        out_specs=[
            pl.BlockSpec((bn, _C), lambda i, k: (i, 0)),
            pl.BlockSpec((2, _C), lambda i, k: (0, 0)),
        ],
        out_shape=[
            jax.ShapeDtypeStruct((_NP, _C), _f32),
            jax.ShapeDtypeStruct((2, _C), _f32),
        ],
    )(num_p, num_p, den_p, den_p, b.reshape(1, _C))


def _tk_final2(y, sums, g, beta, res):
    bn = 1000

    def body(y_ref, s_ref, g_ref, be_ref, r_ref, o_ref):
        mu = s_ref[0] / float(_N)
        var = s_ref[1] / float(_N) - mu * mu
        inv = lax.rsqrt(var + _EPS)
        z = (y_ref[...] - mu) * inv * g_ref[...] + be_ref[...]
        z = jnp.where(z > 0, z, jnp.exp(z) - 1.0)
        o_ref[...] = z + r_ref[...]

    return pl.pallas_call(
        body,
        grid=(_N // bn,),
        in_specs=[
            pl.BlockSpec((bn, _C), lambda i: (i, 0)),
            pl.BlockSpec((2, _C), lambda i: (0, 0)),
            pl.BlockSpec((1, _C), lambda i: (0, 0)),
            pl.BlockSpec((1, _C), lambda i: (0, 0)),
            pl.BlockSpec((bn, _C), lambda i: (i, 0)),
        ],
        out_specs=pl.BlockSpec((bn, _C), lambda i: (i, 0)),
        out_shape=jax.ShapeDtypeStruct((_N, _C), _f32),
    )(y, sums, g.reshape(1, _C), beta.reshape(1, _C), res)


def _sck1(asrc, adst, aef, srcp, dstp, zeros8):
    mesh = plsc.VectorSubcoreMesh(core_axis_name="c", subcore_axis_name="s")

    @functools.partial(
        pl.kernel,
        out_type=[
            jax.ShapeDtypeStruct((_EP, _HP), _f32),
            jax.ShapeDtypeStruct((2 * _NP, _HP), _f32),
        ],
        mesh=mesh,
        compiler_params=pltpu.CompilerParams(use_tc_tiling_on_sc=False),
        scratch_types=[
            pltpu.VMEM((_B,), _i32),
            pltpu.VMEM((_B,), _i32),
            pltpu.VMEM((_B, _HP), _f32),
            pltpu.VMEM((_B, _HP), _f32),
            pltpu.VMEM((_B, _HP), _f32),
            pltpu.VMEM((_B, _HP), _f32),
            pltpu.VMEM_SHARED((_NP, _HP), _f32),
            pltpu.SemaphoreType.DMA,
            pltpu.SemaphoreType.DMA,
        ],
    )
    def k1(as_hbm, ad_hbm, ae_hbm, src_hbm, dst_hbm, z8_hbm,
           w_hbm, den_hbm, srcv, dstv, gsv, gdv, aev, wv, densh, sem1, sem2):
        cid = lax.axis_index("c")
        sid = lax.axis_index("s")
        wid = cid * 16 + sid
        pltpu.sync_copy(z8_hbm.at[pl.ds(sid * _STR, _STR)],
                        densh.at[pl.ds(sid * _STR, _STR)])
        plsc.subcore_barrier()

        def blk(bi, carry):
            e0 = wid * _EW + bi * _B
            pltpu.sync_copy(src_hbm.at[pl.ds(e0, _B)], srcv)
            pltpu.sync_copy(dst_hbm.at[pl.ds(e0, _B)], dstv)
            cp1 = pltpu.async_copy(as_hbm.at[srcv], gsv, sem1)
            cp2 = pltpu.async_copy(ad_hbm.at[dstv], gdv, sem2)
            pltpu.sync_copy(ae_hbm.at[pl.ds(e0, _B)], aev)
            cp1.wait()
            cp2.wait()

            def edge(e, c2):
                a = gsv[e, :] + gdv[e, :] + aev[e, :]
                a = jnp.where(a >= 0, a, _NEG * a)
                wv[e, :] = jnp.exp(a)
                return c2

            lax.fori_loop(0, _B, edge, 0, unroll=4)
            pltpu.sync_copy(wv, w_hbm.at[pl.ds(e0, _B)])
            pltpu.sync_copy(wv, densh.at[dstv], add=True)
            return carry

        lax.fori_loop(0, _NBLK, blk, 0)
        plsc.subcore_barrier()
        pltpu.sync_copy(densh.at[pl.ds(sid * _STR, _STR)],
                        den_hbm.at[pl.ds(cid * _NP + sid * _STR, _STR)])

    return k1(asrc, adst, aef, srcp, dstp, zeros8)


def _sck2(h, xw_flat, srcp, dstp, w, zeros128):
    mesh = plsc.VectorSubcoreMesh(core_axis_name="c", subcore_axis_name="s")

    @functools.partial(
        pl.kernel,
        out_type=jax.ShapeDtypeStruct((2 * h * _NP, _C), _f32),
        mesh=mesh,
        compiler_params=pltpu.CompilerParams(use_tc_tiling_on_sc=False),
        scratch_types=[
            pltpu.VMEM((_B,), _i32),
            pltpu.VMEM((_B,), _i32),
            pltpu.VMEM((_B,), _i32),
            pltpu.VMEM((_B, _HP), _f32),
            pltpu.VMEM((_B, _C), _f32),
            pltpu.VMEM_SHARED((_NP, _C), _f32),
            pltpu.SemaphoreType.DMA,
        ],
    )
    def k2(xw_hbm, src_hbm, dst_hbm, w_hbm, z_hbm, num_hbm,
           srcv, dstv, idxv, wv, rowsv, numsh, sem):
        cid = lax.axis_index("c")
        sid = lax.axis_index("s")
        wid = cid * 16 + sid

        def pass_k(k, carry):
            pltpu.sync_copy(z_hbm.at[pl.ds(sid * _STR, _STR)],
                            numsh.at[pl.ds(sid * _STR, _STR)])
            plsc.subcore_barrier()

            def blk(bi, c2):
                e0 = wid * _EW + bi * _B
                pltpu.sync_copy(src_hbm.at[pl.ds(e0, _B)], srcv)
                pltpu.sync_copy(dst_hbm.at[pl.ds(e0, _B)], dstv)
                pltpu.sync_copy(w_hbm.at[pl.ds(e0, _B)], wv)

                def gidx(g, c3):
                    s = srcv[pl.ds(g * 16, 16)]
                    idxv[pl.ds(g * 16, 16)] = s * h + k
                    return c3

                lax.fori_loop(0, _B // 16, gidx, 0, unroll=4)
                pltpu.async_copy(xw_hbm.at[idxv], rowsv, sem).wait()
                kvec = jnp.full((16, 1), k, _i32)
                dnums = lax.GatherDimensionNumbers(
                    offset_dims=(), collapsed_slice_dims=(0,),
                    start_index_map=(0,))

                def edge(e, c3):
                    wj = lax.gather(
                        wv[e, :], kvec, dnums, slice_sizes=(1,),
                        mode=lax.GatherScatterMode.PROMISE_IN_BOUNDS)

                    def col(c, c4):
                        sl = pl.ds(c * 16, 16)
                        rowsv[e, sl] = rowsv[e, sl] * wj
                        return c4

                    lax.fori_loop(0, _C // 16, col, 0, unroll=8)
                    return c3

                lax.fori_loop(0, _B, edge, 0)
                pltpu.sync_copy(rowsv, numsh.at[dstv], add=True)
                return c2

            lax.fori_loop(0, _NBLK, blk, 0)
            plsc.subcore_barrier()
            pltpu.sync_copy(
                numsh.at[pl.ds(sid * _STR, _STR)],
                num_hbm.at[pl.ds((cid * h + k) * _NP + sid * _STR, _STR)])
            plsc.subcore_barrier()
            return carry

        lax.fori_loop(0, h, pass_k, 0)

    return k2(xw_flat, srcp, dstp, w, zeros128)


def kernel(x, edge_index, edge_attr, params):
    loop = jnp.arange(_N, dtype=edge_index.dtype)
    pad = jnp.zeros((_EP - _E1,), edge_index.dtype)
    srcp = jnp.concatenate([edge_index[0], loop, pad]).astype(_i32)
    dstp = jnp.concatenate([edge_index[1], loop, pad]).astype(_i32)
    zeros8 = jnp.zeros((_NP, _HP), _f32)
    zeros128 = jnp.zeros((_NP, _C), _f32)

    hcur = _tk_in(x, params['in_W'], params['in_b'])
    for i, h in enumerate(_HEADS):
        res = hcur
        xw, asrc, adst = _tk_proj(h, hcur, params['W%d' % i],
                                  params['as%d' % i], params['ad%d' % i])
        ae, _es, aeself = _tk_edge(h, edge_attr, params['We%d' % i],
                                   params['ae%d' % i])
        aef = jnp.concatenate([
            ae,
            jnp.broadcast_to(aeself, (_N, _HP)),
            jnp.full((_EP - _E1, _HP), -1e30, _f32),
        ], axis=0)
        w, den_p = _sck1(asrc, adst, aef, srcp, dstp, zeros8)
        num_p = _sck2(h, xw.reshape(_N * h, _C), srcp, dstp, w, zeros128)
        y, sums = _tk_final1(h, num_p, den_p, params['b%d' % i])
        hcur = _tk_final2(y, sums, params['g%d' % i],
                          params['beta%d' % i], res)
    return hcur
